# baseline (speedup 1.0000x reference)
import functools
import math

import jax
import jax.numpy as jnp
from jax import lax
from jax.experimental import pallas as pl
from jax.experimental.pallas import tpu as pltpu
from jax.experimental.pallas import tpu_sc as plsc

C_SPARSITY = 0.8
_LANES = 16


def _pool_body(x_ref, sum_ref, max_ref):
    xb = x_ref[...]
    s = jnp.sum(xb, axis=(3, 4))
    m = jnp.max(xb, axis=(3, 4))
    sum_ref[0, 0, 0, :] = jnp.sum(s, axis=(0, 1))
    max_ref[0, 0, 0, :] = jnp.max(m, axis=(0, 1))


def _make_sc_mask_kernel(B, C, R, n_pool, k):
    NCH = C // _LANES
    inv_n = 1.0 / n_pool
    mesh = plsc.VectorSubcoreMesh(core_axis_name="c", subcore_axis_name="s")

    @functools.partial(
        pl.kernel,
        out_type=jax.ShapeDtypeStruct((B, C), jnp.float32),
        mesh=mesh,
        scratch_types=[
            pltpu.VMEM((C,), jnp.float32),
            pltpu.VMEM((C,), jnp.float32),
            pltpu.VMEM((R, C), jnp.float32),
            pltpu.VMEM((R, C), jnp.float32),
            pltpu.VMEM((C,), jnp.float32),
        ],
    )
    def body(sums_hbm, maxs_hbm, w1_hbm, w2t_hbm, out_hbm,
             sv, mv, w1v, w2tv, cv):
        cid = lax.axis_index("c")
        sid = lax.axis_index("s")

        @pl.when((cid == 0) & (sid < B))
        def _():
            b = sid
            pltpu.sync_copy(sums_hbm.at[b], sv)
            pltpu.sync_copy(maxs_hbm.at[b], mv)
            pltpu.sync_copy(w1_hbm, w1v)
            pltpu.sync_copy(w2t_hbm, w2tv)

            iota = lax.broadcasted_iota(jnp.int32, (_LANES,), 0)

            def gat(v, idx):
                return v.at[idx].get(mode="promise_in_bounds")

            def splat_sum(v):
                for d in (1, 2, 4, 8):
                    v = v + gat(v, iota ^ d)
                return v

            def prefix_sum(v):
                for d in (1, 2, 4, 8):
                    shifted = gat(v, jnp.maximum(iota - d, 0))
                    v = v + jnp.where(iota >= d, shifted, jnp.zeros_like(v))
                return v

            av = [sv[pl.ds(j * _LANES, _LANES)] * inv_n for j in range(NCH)]
            mx = [mv[pl.ds(j * _LANES, _LANES)] for j in range(NCH)]

            g = []
            for r in range(R):
                row0 = w1v[r, pl.ds(0, _LANES)]
                acc_a = av[0] * row0
                acc_m = mx[0] * row0
                for j in range(1, NCH):
                    row = w1v[r, pl.ds(j * _LANES, _LANES)]
                    acc_a = acc_a + av[j] * row
                    acc_m = acc_m + mx[j] * row
                ha = jnp.maximum(splat_sum(acc_a), 0.0)
                hm = jnp.maximum(splat_sum(acc_m), 0.0)
                g.append(ha + hm)

            s = []
            for j in range(NCH):
                accz = g[0] * w2tv[0, pl.ds(j * _LANES, _LANES)]
                for r in range(1, R):
                    accz = accz + g[r] * w2tv[r, pl.ds(j * _LANES, _LANES)]
                s.append(1.0 / (1.0 + jnp.exp(-accz)))

            kvec = jnp.full((_LANES,), k, jnp.int32)

            def count_ge(t):
                cnt = jnp.zeros((_LANES,), jnp.int32)
                for j in range(NCH):
                    cnt = cnt + jnp.where(s[j] >= t, 1, 0)
                return splat_sum(cnt)

            def search_step(_, lohi):
                lo, hi = lohi
                mid = (lo + hi) * 0.5
                take = count_ge(mid) >= kvec
                lo = jnp.where(take, mid, lo)
                hi = jnp.where(take, hi, mid)
                return lo, hi

            lo, hi = lax.fori_loop(
                0, 100, search_step,
                (jnp.zeros((_LANES,), jnp.float32),
                 jnp.full((_LANES,), 2.0, jnp.float32)))

            need = (kvec - count_ge(hi)).astype(jnp.float32)

            last = jnp.full((_LANES,), _LANES - 1, jnp.int32)
            carry = jnp.zeros((_LANES,), jnp.float32)
            for j in range(NCH):
                eq = (s[j] >= lo) & (s[j] < hi)
                eqf = jnp.where(eq, 1.0, 0.0)
                pos = prefix_sum(eqf) + carry
                sel = eq & (pos <= need)
                maskv = (jnp.where(s[j] >= hi, 1.0, 0.0)
                         + jnp.where(sel, 1.0, 0.0))
                cv[pl.ds(j * _LANES, _LANES)] = s[j] * maskv
                carry = gat(pos, last)

            pltpu.sync_copy(cv, out_hbm.at[b])

    return body


def _make_mask_body(n_pool, k):
    def _mask_body(sum_ref, max_ref, w1_ref, w2_ref, coef_ref):
        avg = sum_ref[...] * (1.0 / n_pool)
        mx = max_ref[...]
        w1 = w1_ref[...]
        w2 = w2_ref[...]

        def mlp(v):
            h = jax.lax.dot_general(v, w1, (((1,), (1,)), ((), ())),
                                    preferred_element_type=jnp.float32)
            h = jnp.maximum(h, 0.0)
            return jax.lax.dot_general(h, w2, (((1,), (1,)), ((), ())),
                                       preferred_element_type=jnp.float32)

        s = jax.nn.sigmoid(mlp(avg) + mlp(mx))
        c = s.shape[1]
        si = s[:, :, None]
        sj = s[:, None, :]
        gt = jnp.sum(jnp.where(sj > si, 1.0, 0.0), axis=2)
        ii = jax.lax.broadcasted_iota(jnp.int32, (1, c, c), 1)
        jj = jax.lax.broadcasted_iota(jnp.int32, (1, c, c), 2)
        eq_lower = jnp.sum(
            jnp.where((sj == si) & (jj < ii), 1.0, 0.0), axis=2)
        rank = gt + eq_lower
        mask = jnp.where(rank < float(k), 1.0, 0.0)
        coef_ref[...] = s * mask
    return _mask_body


def _make_mul_body(cb, hw):
    h, w = hw

    def _mul_body(x_hbm, coef_ref, o_ref, buf, sems):
        b = pl.program_id(0)
        f = pl.program_id(1)
        i = pl.program_id(2)
        nf = pl.num_programs(1)
        ni = pl.num_programs(2)
        n = (b * nf + f) * ni + i
        total = pl.num_programs(0) * nf * ni

        def issue(m):
            bm = m // (nf * ni)
            fm = (m // ni) % nf
            im = m % ni
            slot = m % 2
            for j in range(cb):
                co = coef_ref[bm, im * cb + j]

                @pl.when(co != 0.0)
                def _():
                    pltpu.make_async_copy(
                        x_hbm.at[bm, fm, im * cb + j],
                        buf.at[slot, j],
                        sems.at[slot, j],
                    ).start()

        @pl.when(n == 0)
        def _():
            issue(n)

        @pl.when(n + 1 < total)
        def _():
            issue(n + 1)

        slot = n % 2
        for j in range(cb):
            co = coef_ref[b, i * cb + j]

            @pl.when(co != 0.0)
            def _():
                pltpu.make_async_copy(
                    x_hbm.at[b, f, i * cb + j],
                    buf.at[slot, j],
                    sems.at[slot, j],
                ).wait()
                o_ref[0, 0, j] = buf[slot, j] * co

            @pl.when(co == 0.0)
            def _():
                o_ref[0, 0, j] = jnp.zeros((h, w), jnp.float32)
    return _mul_body


def kernel(x, W1, W2):
    B, F, C, H, W = x.shape
    k = int(math.ceil(C * C_SPARSITY))

    CB1 = 16
    NC1 = C // CB1
    sums, maxs = pl.pallas_call(
        _pool_body,
        grid=(B, NC1),
        in_specs=[pl.BlockSpec((1, F, CB1, H, W),
                               lambda b, i: (b, 0, i, 0, 0))],
        out_specs=[pl.BlockSpec((1, 1, 1, CB1), lambda b, i: (b, i, 0, 0)),
                   pl.BlockSpec((1, 1, 1, CB1), lambda b, i: (b, i, 0, 0))],
        out_shape=[jax.ShapeDtypeStruct((B, NC1, 1, CB1), jnp.float32),
                   jax.ShapeDtypeStruct((B, NC1, 1, CB1), jnp.float32)],
    )(x)
    sums = sums.reshape(B, C)
    maxs = maxs.reshape(B, C)

    sc_mask = _make_sc_mask_kernel(B, C, W1.shape[0], float(F * H * W), k)
    coef = sc_mask(sums, maxs, W1, W2.T)

    CB3 = 32
    out = pl.pallas_call(
        _make_mul_body(CB3, (H, W)),
        grid=(B, F, C // CB3),
        in_specs=[pl.BlockSpec(memory_space=pltpu.MemorySpace.HBM),
                  pl.BlockSpec(memory_space=pltpu.SMEM)],
        out_specs=pl.BlockSpec((1, 1, CB3, H, W),
                               lambda b, f, i: (b, f, i, 0, 0)),
        out_shape=jax.ShapeDtypeStruct(x.shape, x.dtype),
        scratch_shapes=[pltpu.VMEM((2, CB3, H, W), jnp.float32),
                        pltpu.SemaphoreType.DMA((2, CB3))],
    )(x, coef)
    return out


# speedup vs baseline: 1.0069x; 1.0069x over previous
import functools
import math

import jax
import jax.numpy as jnp
from jax import lax
from jax.experimental import pallas as pl
from jax.experimental.pallas import tpu as pltpu
from jax.experimental.pallas import tpu_sc as plsc

C_SPARSITY = 0.8
_LANES = 16


def _pool_body(x_ref, sum_ref, max_ref):
    xb = x_ref[...]
    s = jnp.sum(xb, axis=(3, 4))
    m = jnp.max(xb, axis=(3, 4))
    sum_ref[0, 0, 0, :] = jnp.sum(s, axis=(0, 1))
    max_ref[0, 0, 0, :] = jnp.max(m, axis=(0, 1))


def _make_sc_mask_kernel(B, C, R, n_pool, k):
    NCH = C // _LANES
    inv_n = 1.0 / n_pool
    mesh = plsc.VectorSubcoreMesh(core_axis_name="c", subcore_axis_name="s")

    @functools.partial(
        pl.kernel,
        out_type=jax.ShapeDtypeStruct((B, C), jnp.float32),
        mesh=mesh,
        scratch_types=[
            pltpu.VMEM((C,), jnp.float32),
            pltpu.VMEM((C,), jnp.float32),
            pltpu.VMEM((R, C), jnp.float32),
            pltpu.VMEM((R, C), jnp.float32),
            pltpu.VMEM((C,), jnp.float32),
            pltpu.SemaphoreType.DMA((4,)),
        ],
    )
    def body(sums_hbm, maxs_hbm, w1_hbm, w2t_hbm, out_hbm,
             sv, mv, w1v, w2tv, cv, dsem):
        cid = lax.axis_index("c")
        sid = lax.axis_index("s")

        @pl.when((cid == 0) & (sid < B))
        def _():
            b = sid
            cps = [pltpu.make_async_copy(sums_hbm.at[b], sv, dsem.at[0]),
                   pltpu.make_async_copy(maxs_hbm.at[b], mv, dsem.at[1]),
                   pltpu.make_async_copy(w1_hbm, w1v, dsem.at[2]),
                   pltpu.make_async_copy(w2t_hbm, w2tv, dsem.at[3])]
            for cp in cps:
                cp.start()
            for cp in cps:
                cp.wait()

            iota = lax.broadcasted_iota(jnp.int32, (_LANES,), 0)

            def gat(v, idx):
                return v.at[idx].get(mode="promise_in_bounds")

            def splat_sum(v):
                for d in (1, 2, 4, 8):
                    v = v + gat(v, iota ^ d)
                return v

            def prefix_sum(v):
                for d in (1, 2, 4, 8):
                    shifted = gat(v, jnp.maximum(iota - d, 0))
                    v = v + jnp.where(iota >= d, shifted, jnp.zeros_like(v))
                return v

            av = [sv[pl.ds(j * _LANES, _LANES)] * inv_n for j in range(NCH)]
            mx = [mv[pl.ds(j * _LANES, _LANES)] for j in range(NCH)]

            g = []
            for r in range(R):
                row0 = w1v[r, pl.ds(0, _LANES)]
                acc_a = av[0] * row0
                acc_m = mx[0] * row0
                for j in range(1, NCH):
                    row = w1v[r, pl.ds(j * _LANES, _LANES)]
                    acc_a = acc_a + av[j] * row
                    acc_m = acc_m + mx[j] * row
                ha = jnp.maximum(splat_sum(acc_a), 0.0)
                hm = jnp.maximum(splat_sum(acc_m), 0.0)
                g.append(ha + hm)

            s = []
            for j in range(NCH):
                accz = g[0] * w2tv[0, pl.ds(j * _LANES, _LANES)]
                for r in range(1, R):
                    accz = accz + g[r] * w2tv[r, pl.ds(j * _LANES, _LANES)]
                s.append(1.0 / (1.0 + jnp.exp(-accz)))

            kvec = jnp.full((_LANES,), k, jnp.int32)

            def count_ge(t):
                cnt = jnp.zeros((_LANES,), jnp.int32)
                for j in range(NCH):
                    cnt = cnt + jnp.where(s[j] >= t, 1, 0)
                return splat_sum(cnt)

            def search_step(_, lohi):
                lo, hi = lohi
                mid = (lo + hi) * 0.5
                take = count_ge(mid) >= kvec
                lo = jnp.where(take, mid, lo)
                hi = jnp.where(take, hi, mid)
                return lo, hi

            lo, hi = lax.fori_loop(
                0, 48, search_step,
                (jnp.zeros((_LANES,), jnp.float32),
                 jnp.full((_LANES,), 2.0, jnp.float32)))

            need = (kvec - count_ge(hi)).astype(jnp.float32)

            last = jnp.full((_LANES,), _LANES - 1, jnp.int32)
            carry = jnp.zeros((_LANES,), jnp.float32)
            for j in range(NCH):
                eq = (s[j] >= lo) & (s[j] < hi)
                eqf = jnp.where(eq, 1.0, 0.0)
                pos = prefix_sum(eqf) + carry
                sel = eq & (pos <= need)
                maskv = (jnp.where(s[j] >= hi, 1.0, 0.0)
                         + jnp.where(sel, 1.0, 0.0))
                cv[pl.ds(j * _LANES, _LANES)] = s[j] * maskv
                carry = gat(pos, last)

            pltpu.sync_copy(cv, out_hbm.at[b])

    return body


def _make_mask_body(n_pool, k):
    def _mask_body(sum_ref, max_ref, w1_ref, w2_ref, coef_ref):
        avg = sum_ref[...] * (1.0 / n_pool)
        mx = max_ref[...]
        w1 = w1_ref[...]
        w2 = w2_ref[...]

        def mlp(v):
            h = jax.lax.dot_general(v, w1, (((1,), (1,)), ((), ())),
                                    preferred_element_type=jnp.float32)
            h = jnp.maximum(h, 0.0)
            return jax.lax.dot_general(h, w2, (((1,), (1,)), ((), ())),
                                       preferred_element_type=jnp.float32)

        s = jax.nn.sigmoid(mlp(avg) + mlp(mx))
        c = s.shape[1]
        si = s[:, :, None]
        sj = s[:, None, :]
        gt = jnp.sum(jnp.where(sj > si, 1.0, 0.0), axis=2)
        ii = jax.lax.broadcasted_iota(jnp.int32, (1, c, c), 1)
        jj = jax.lax.broadcasted_iota(jnp.int32, (1, c, c), 2)
        eq_lower = jnp.sum(
            jnp.where((sj == si) & (jj < ii), 1.0, 0.0), axis=2)
        rank = gt + eq_lower
        mask = jnp.where(rank < float(k), 1.0, 0.0)
        coef_ref[...] = s * mask
    return _mask_body


def _make_mul_body(cb, hw):
    h, w = hw

    def _mul_body(x_hbm, coef_ref, o_ref, buf, sems):
        b = pl.program_id(0)
        f = pl.program_id(1)
        i = pl.program_id(2)
        nf = pl.num_programs(1)
        ni = pl.num_programs(2)
        n = (b * nf + f) * ni + i
        total = pl.num_programs(0) * nf * ni

        def issue(m):
            bm = m // (nf * ni)
            fm = (m // ni) % nf
            im = m % ni
            slot = m % 2
            for j in range(cb):
                co = coef_ref[bm, im * cb + j]

                @pl.when(co != 0.0)
                def _():
                    pltpu.make_async_copy(
                        x_hbm.at[bm, fm, im * cb + j],
                        buf.at[slot, j],
                        sems.at[slot, j],
                    ).start()

        @pl.when(n == 0)
        def _():
            issue(n)

        @pl.when(n + 1 < total)
        def _():
            issue(n + 1)

        slot = n % 2
        for j in range(cb):
            co = coef_ref[b, i * cb + j]

            @pl.when(co != 0.0)
            def _():
                pltpu.make_async_copy(
                    x_hbm.at[b, f, i * cb + j],
                    buf.at[slot, j],
                    sems.at[slot, j],
                ).wait()
                o_ref[0, 0, j] = buf[slot, j] * co

            @pl.when(co == 0.0)
            def _():
                o_ref[0, 0, j] = jnp.zeros((h, w), jnp.float32)
    return _mul_body


def kernel(x, W1, W2):
    B, F, C, H, W = x.shape
    k = int(math.ceil(C * C_SPARSITY))

    CB1 = 16
    NC1 = C // CB1
    sums, maxs = pl.pallas_call(
        _pool_body,
        grid=(B, NC1),
        in_specs=[pl.BlockSpec((1, F, CB1, H, W),
                               lambda b, i: (b, 0, i, 0, 0))],
        out_specs=[pl.BlockSpec((1, 1, 1, CB1), lambda b, i: (b, i, 0, 0)),
                   pl.BlockSpec((1, 1, 1, CB1), lambda b, i: (b, i, 0, 0))],
        out_shape=[jax.ShapeDtypeStruct((B, NC1, 1, CB1), jnp.float32),
                   jax.ShapeDtypeStruct((B, NC1, 1, CB1), jnp.float32)],
    )(x)
    sums = sums.reshape(B, C)
    maxs = maxs.reshape(B, C)

    sc_mask = _make_sc_mask_kernel(B, C, W1.shape[0], float(F * H * W), k)
    coef = sc_mask(sums, maxs, W1, W2.T)

    CB3 = 32
    out = pl.pallas_call(
        _make_mul_body(CB3, (H, W)),
        grid=(B, F, C // CB3),
        in_specs=[pl.BlockSpec(memory_space=pltpu.MemorySpace.HBM),
                  pl.BlockSpec(memory_space=pltpu.SMEM)],
        out_specs=pl.BlockSpec((1, 1, CB3, H, W),
                               lambda b, f, i: (b, f, i, 0, 0)),
        out_shape=jax.ShapeDtypeStruct(x.shape, x.dtype),
        scratch_shapes=[pltpu.VMEM((2, CB3, H, W), jnp.float32),
                        pltpu.SemaphoreType.DMA((2, CB3))],
    )(x, coef)
    return out


# speedup vs baseline: 1.0072x; 1.0003x over previous
import functools
import math

import jax
import jax.numpy as jnp
from jax import lax
from jax.experimental import pallas as pl
from jax.experimental.pallas import tpu as pltpu
from jax.experimental.pallas import tpu_sc as plsc

C_SPARSITY = 0.8
_LANES = 16


def _pool_body(x_ref, sum_ref, max_ref):
    xb = x_ref[...]
    s = jnp.sum(xb, axis=(3, 4))
    m = jnp.max(xb, axis=(3, 4))
    sum_ref[0, 0, 0, :] = jnp.sum(s, axis=(0, 1))
    max_ref[0, 0, 0, :] = jnp.max(m, axis=(0, 1))


def _make_sc_mask_kernel(B, C, R, n_pool, k):
    NCH = C // _LANES
    inv_n = 1.0 / n_pool
    mesh = plsc.VectorSubcoreMesh(core_axis_name="c", subcore_axis_name="s")

    @functools.partial(
        pl.kernel,
        out_type=jax.ShapeDtypeStruct((B, C), jnp.float32),
        mesh=mesh,
        scratch_types=[
            pltpu.VMEM((C,), jnp.float32),
            pltpu.VMEM((C,), jnp.float32),
            pltpu.VMEM((R, C), jnp.float32),
            pltpu.VMEM((R, C), jnp.float32),
            pltpu.VMEM((C,), jnp.float32),
            pltpu.SemaphoreType.DMA((4,)),
        ],
    )
    def body(sums_hbm, maxs_hbm, w1_hbm, w2t_hbm, out_hbm,
             sv, mv, w1v, w2tv, cv, dsem):
        cid = lax.axis_index("c")
        sid = lax.axis_index("s")

        @pl.when((cid == 0) & (sid < B))
        def _():
            b = sid
            cps = [pltpu.make_async_copy(sums_hbm.at[b], sv, dsem.at[0]),
                   pltpu.make_async_copy(maxs_hbm.at[b], mv, dsem.at[1]),
                   pltpu.make_async_copy(w1_hbm, w1v, dsem.at[2]),
                   pltpu.make_async_copy(w2t_hbm, w2tv, dsem.at[3])]
            for cp in cps:
                cp.start()
            for cp in cps:
                cp.wait()

            iota = lax.broadcasted_iota(jnp.int32, (_LANES,), 0)

            def gat(v, idx):
                return v.at[idx].get(mode="promise_in_bounds")

            def splat_sum(v):
                for d in (1, 2, 4, 8):
                    v = v + gat(v, iota ^ d)
                return v

            def prefix_sum(v):
                for d in (1, 2, 4, 8):
                    shifted = gat(v, jnp.maximum(iota - d, 0))
                    v = v + jnp.where(iota >= d, shifted, jnp.zeros_like(v))
                return v

            av = [sv[pl.ds(j * _LANES, _LANES)] * inv_n for j in range(NCH)]
            mx = [mv[pl.ds(j * _LANES, _LANES)] for j in range(NCH)]

            g = []
            for r in range(R):
                row0 = w1v[r, pl.ds(0, _LANES)]
                acc_a = av[0] * row0
                acc_m = mx[0] * row0
                for j in range(1, NCH):
                    row = w1v[r, pl.ds(j * _LANES, _LANES)]
                    acc_a = acc_a + av[j] * row
                    acc_m = acc_m + mx[j] * row
                ha = jnp.maximum(splat_sum(acc_a), 0.0)
                hm = jnp.maximum(splat_sum(acc_m), 0.0)
                g.append(ha + hm)

            s = []
            for j in range(NCH):
                accz = g[0] * w2tv[0, pl.ds(j * _LANES, _LANES)]
                for r in range(1, R):
                    accz = accz + g[r] * w2tv[r, pl.ds(j * _LANES, _LANES)]
                s.append(1.0 / (1.0 + jnp.exp(-accz)))

            kvec = jnp.full((_LANES,), k, jnp.int32)

            def count_ge(t):
                cnt = jnp.zeros((_LANES,), jnp.int32)
                for j in range(NCH):
                    cnt = cnt + jnp.where(s[j] >= t, 1, 0)
                return splat_sum(cnt)

            def search_step(_, lohi):
                lo, hi = lohi
                mid = (lo + hi) * 0.5
                take = count_ge(mid) >= kvec
                lo = jnp.where(take, mid, lo)
                hi = jnp.where(take, hi, mid)
                return lo, hi

            lo, hi = lax.fori_loop(
                0, 48, search_step,
                (jnp.zeros((_LANES,), jnp.float32),
                 jnp.full((_LANES,), 2.0, jnp.float32)))

            need = (kvec - count_ge(hi)).astype(jnp.float32)

            last = jnp.full((_LANES,), _LANES - 1, jnp.int32)
            carry = jnp.zeros((_LANES,), jnp.float32)
            for j in range(NCH):
                eq = (s[j] >= lo) & (s[j] < hi)
                eqf = jnp.where(eq, 1.0, 0.0)
                pos = prefix_sum(eqf) + carry
                sel = eq & (pos <= need)
                maskv = (jnp.where(s[j] >= hi, 1.0, 0.0)
                         + jnp.where(sel, 1.0, 0.0))
                cv[pl.ds(j * _LANES, _LANES)] = s[j] * maskv
                carry = gat(pos, last)

            pltpu.sync_copy(cv, out_hbm.at[b])

    return body


def _make_mul_body(cb, hw):
    h, w = hw

    def _mul_body(x_hbm, coef_ref, o_ref, buf, sems):
        b = pl.program_id(0)
        f = pl.program_id(1)
        i = pl.program_id(2)
        nf = pl.num_programs(1)
        ni = pl.num_programs(2)
        n = (b * nf + f) * ni + i
        total = pl.num_programs(0) * nf * ni

        def issue(m):
            bm = m // (nf * ni)
            fm = (m // ni) % nf
            im = m % ni
            slot = m % 2
            for j in range(cb):
                co = coef_ref[bm, im * cb + j]

                @pl.when(co != 0.0)
                def _():
                    pltpu.make_async_copy(
                        x_hbm.at[bm, fm, im * cb + j],
                        buf.at[slot, j],
                        sems.at[slot, j],
                    ).start()

        @pl.when(n == 0)
        def _():
            issue(n)

        @pl.when(n + 1 < total)
        def _():
            issue(n + 1)

        slot = n % 2
        for j in range(cb):
            co = coef_ref[b, i * cb + j]

            @pl.when(co != 0.0)
            def _():
                pltpu.make_async_copy(
                    x_hbm.at[b, f, i * cb + j],
                    buf.at[slot, j],
                    sems.at[slot, j],
                ).wait()
                o_ref[0, 0, j] = buf[slot, j] * co

            @pl.when(co == 0.0)
            def _():
                o_ref[0, 0, j] = jnp.zeros((h, w), jnp.float32)
    return _mul_body


def kernel(x, W1, W2):
    B, F, C, H, W = x.shape
    k = int(math.ceil(C * C_SPARSITY))

    CB1 = 16
    NC1 = C // CB1
    sums, maxs = pl.pallas_call(
        _pool_body,
        grid=(B, NC1),
        in_specs=[pl.BlockSpec((1, F, CB1, H, W),
                               lambda b, i: (b, 0, i, 0, 0))],
        out_specs=[pl.BlockSpec((1, 1, 1, CB1), lambda b, i: (b, i, 0, 0)),
                   pl.BlockSpec((1, 1, 1, CB1), lambda b, i: (b, i, 0, 0))],
        out_shape=[jax.ShapeDtypeStruct((B, NC1, 1, CB1), jnp.float32),
                   jax.ShapeDtypeStruct((B, NC1, 1, CB1), jnp.float32)],
    )(x)
    sums = sums.reshape(B, C)
    maxs = maxs.reshape(B, C)

    sc_mask = _make_sc_mask_kernel(B, C, W1.shape[0], float(F * H * W), k)
    coef = sc_mask(sums, maxs, W1, W2.T)

    CB3 = 32
    out = pl.pallas_call(
        _make_mul_body(CB3, (H, W)),
        grid=(B, F, C // CB3),
        in_specs=[pl.BlockSpec(memory_space=pltpu.MemorySpace.HBM),
                  pl.BlockSpec(memory_space=pltpu.SMEM)],
        out_specs=pl.BlockSpec((1, 1, CB3, H, W),
                               lambda b, f, i: (b, f, i, 0, 0)),
        out_shape=jax.ShapeDtypeStruct(x.shape, x.dtype),
        scratch_shapes=[pltpu.VMEM((2, CB3, H, W), jnp.float32),
                        pltpu.SemaphoreType.DMA((2, CB3))],
    )(x, coef)
    return out
